# baseline (speedup 1.0000x reference)
import functools

import jax
import jax.numpy as jnp
from jax import lax
from jax.experimental import pallas as pl
from jax.experimental.pallas import tpu as pltpu
from jax.experimental.pallas import tpu_sc as plsc

N = 10000
E = 320000
D_IN = 128
D_HID = 8
N_CLASSES = 10
PAD = 16
SW = 12
ULANE = SW - 1
NEG = 0.2

NC = 2
NS = 16
NW = NC * NS
NP = 10240
RPT = NP // NS
EPAD = NW * NP
EPW = EPAD // NW
BLK = 128
NBLK = EPW // BLK
NPAIR = NBLK // 2


def _lrelu(t):
    return jnp.maximum(t, NEG * t)


def _dense1_body(x_ref, w_ref, asrc_ref, adst_ref, hp_ref, as_ref, ad_ref, m_ref):
    h = jnp.dot(x_ref[...], w_ref[...], preferred_element_type=jnp.float32)
    a_s = jnp.sum(h * asrc_ref[...], axis=1)
    a_d = jnp.sum(h * adst_ref[...], axis=1)
    m = _lrelu(jnp.max(a_s) + a_d)
    hp = jnp.concatenate([h, jnp.zeros((N, PAD - D_HID), jnp.float32)], axis=1)
    hp_ref[...] = jnp.concatenate([hp, jnp.zeros((NP - N, PAD), jnp.float32)], axis=0)
    zpad = jnp.zeros((NP - N,), jnp.float32)
    as_ref[...] = jnp.concatenate([a_s, zpad])
    ad_ref[...] = jnp.concatenate([a_d, zpad])
    m_ref[...] = jnp.concatenate([m, zpad])


def _dense2_body(acc_ref, b1_ref, w2_ref, asrc_ref, adst_ref, hp_ref, as_ref, ad_ref, m_ref):
    acc = acc_ref[0] + acc_ref[1]
    o = acc[:N, :D_HID] / (acc[:N, ULANE:ULANE + 1] + 1e-16) + b1_ref[...]
    x2 = jnp.maximum(o, 0.0)
    h = jnp.dot(x2, w2_ref[...], preferred_element_type=jnp.float32)
    a_s = jnp.sum(h * asrc_ref[...], axis=1)
    a_d = jnp.sum(h * adst_ref[...], axis=1)
    m = _lrelu(jnp.max(a_s) + a_d)
    hp = jnp.concatenate([h, jnp.zeros((N, PAD - N_CLASSES), jnp.float32)], axis=1)
    hp_ref[...] = jnp.concatenate([hp, jnp.zeros((NP - N, PAD), jnp.float32)], axis=0)
    zpad = jnp.zeros((NP - N,), jnp.float32)
    as_ref[...] = jnp.concatenate([a_s, zpad])
    ad_ref[...] = jnp.concatenate([a_d, zpad])
    m_ref[...] = jnp.concatenate([m, zpad])


def _final_body(acc_ref, b2_ref, out_ref):
    acc = acc_ref[0] + acc_ref[1]
    h = acc[:N, :N_CLASSES] / (acc[:N, ULANE:ULANE + 1] + 1e-16) + b2_ref[...]
    m = jnp.max(h, axis=1, keepdims=True)
    lse = jnp.log(jnp.sum(jnp.exp(h - m), axis=1, keepdims=True))
    out_ref[...] = h - m - lse


_dense1 = pl.pallas_call(
    _dense1_body,
    out_shape=[jax.ShapeDtypeStruct((NP, PAD), jnp.float32),
               jax.ShapeDtypeStruct((NP,), jnp.float32),
               jax.ShapeDtypeStruct((NP,), jnp.float32),
               jax.ShapeDtypeStruct((NP,), jnp.float32)])

_dense2 = pl.pallas_call(
    _dense2_body,
    out_shape=[jax.ShapeDtypeStruct((NP, PAD), jnp.float32),
               jax.ShapeDtypeStruct((NP,), jnp.float32),
               jax.ShapeDtypeStruct((NP,), jnp.float32),
               jax.ShapeDtypeStruct((NP,), jnp.float32)])

_final = pl.pallas_call(
    _final_body,
    out_shape=jax.ShapeDtypeStruct((N, N_CLASSES), jnp.float32))


_sc_mesh = plsc.VectorSubcoreMesh(core_axis_name="c", subcore_axis_name="s")


@functools.partial(
    pl.kernel,
    out_type=jax.ShapeDtypeStruct((NC, NP, SW), jnp.float32),
    mesh=_sc_mesh,
    compiler_params=pltpu.CompilerParams(needs_layout_passes=False,
                                         use_tc_tiling_on_sc=False),
    scratch_types=[
        pltpu.VMEM_SHARED((NP, SW), jnp.float32),
        pltpu.VMEM((RPT, SW), jnp.float32),
        pltpu.VMEM((EPW,), jnp.int32),
        pltpu.VMEM((EPW,), jnp.int32),
        pltpu.VMEM((NP,), jnp.float32),
        pltpu.VMEM((NP,), jnp.float32),
        pltpu.VMEM((NP,), jnp.float32),
        pltpu.VMEM((BLK, PAD), jnp.float32),
        pltpu.VMEM((BLK, PAD), jnp.float32),
        pltpu.VMEM((BLK, SW), jnp.float32),
        pltpu.VMEM((BLK, SW), jnp.float32),
        pltpu.VMEM((BLK,), jnp.float32),
        pltpu.VMEM((BLK,), jnp.int32),
        pltpu.VMEM((BLK,), jnp.int32),
        pltpu.VMEM((BLK,), jnp.int32),
        pltpu.VMEM((BLK,), jnp.int32),
        pltpu.VMEM((BLK,), jnp.int32),
        pltpu.VMEM((BLK,), jnp.int32),
        pltpu.SemaphoreType.DMA,
        pltpu.SemaphoreType.DMA,
        pltpu.SemaphoreType.DMA,
        pltpu.SemaphoreType.DMA,
    ])
def _edge_sc(src_hbm, dst_hbm, as_hbm, ad_hbm, m_hbm, hp_hbm, out_hbm,
             acc_sh, stage, srcv, dstv, asv, adv, mv,
             hr0, hr1, sr0, sr1, ublk,
             sidx0, sidx1, didx0, didx1, didxs0, didxs1,
             sg0, sg1, ss0, ss1):
    cid = lax.axis_index("c")
    sid = lax.axis_index("s")
    wid = cid * jnp.int32(NS) + sid
    r0 = sid * jnp.int32(RPT)

    sidx = (sidx0, sidx1)
    didx = (didx0, didx1)
    didxs = (didxs0, didxs1)
    hr = (hr0, hr1)
    sr = (sr0, sr1)
    sg = (sg0, sg1)
    ss = (ss0, ss1)
    lane = lax.iota(jnp.int32, 16)
    mask_u = lane == ULANE
    mask_sw = lane < jnp.int32(SW)
    zvec = jnp.zeros((16,), jnp.float32)

    for r in range(RPT):
        plsc.store_scatter(stage, [jnp.full((16,), r, jnp.int32), lane],
                           zvec, mask=mask_sw)
    pltpu.sync_copy(stage, acc_sh.at[pl.ds(r0, RPT)])

    pltpu.sync_copy(as_hbm, asv)
    pltpu.sync_copy(ad_hbm, adv)
    pltpu.sync_copy(m_hbm, mv)
    e0 = wid * jnp.int32(EPW)
    pltpu.sync_copy(src_hbm.at[pl.ds(e0, EPW)], srcv)
    pltpu.sync_copy(dst_hbm.at[pl.ds(e0, EPW)], dstv)

    plsc.subcore_barrier()

    def _prep(p, base):
        for v in range(BLK // 16):
            sidx[p][pl.ds(v * 16, 16)] = srcv[pl.ds(base + v * 16, 16)]
            didx[p][pl.ds(v * 16, 16)] = dstv[pl.ds(base + v * 16, 16)]

    def _issue_gather(p):
        pltpu.async_copy(hp_hbm.at[sidx[p]], hr[p], sg[p])

    def _wait_gather(p):
        pltpu.make_async_copy(hp_hbm.at[sidx[p]], hr[p], sg[p]).wait()

    def _issue_scatter(p):
        pltpu.async_copy(sr[p], acc_sh.at[didxs[p]], ss[p], add=True)

    def _wait_scatter(p):
        pltpu.make_async_copy(sr[p], acc_sh.at[didxs[p]], ss[p]).wait()

    def _scale_rows(p):
        for v in range(BLK // 16):
            sv = sidx[p][pl.ds(v * 16, 16)]
            dv = didx[p][pl.ds(v * 16, 16)]
            a = plsc.load_gather(asv, [sv]) + plsc.load_gather(adv, [dv])
            a = _lrelu(a) - plsc.load_gather(mv, [dv])
            uvec = jnp.exp(a)
            for i in range(16):
                r = v * 16 + i
                ubc = uvec[jnp.full((16,), i, jnp.int32)]
                val = jnp.where(mask_u, ubc, hr[p][r, :] * ubc)
                plsc.store_scatter(sr[p], [jnp.full((16,), r, jnp.int32), lane],
                                   val, mask=mask_sw)

    def _copy_didx(p):
        for v in range(BLK // 16):
            didxs[p][pl.ds(v * 16, 16)] = didx[p][pl.ds(v * 16, 16)]

    def _block(p, base, first, last):
        _wait_gather(p)
        if not first:
            _wait_scatter(p)
        _scale_rows(p)
        _copy_didx(p)
        _issue_scatter(p)
        if not last:
            _prep(p, base + jnp.int32(2 * BLK))
            _issue_gather(p)

    _prep(0, jnp.int32(0))
    _issue_gather(0)
    _prep(1, jnp.int32(BLK))
    _issue_gather(1)
    _block(0, jnp.int32(0), True, False)
    _block(1, jnp.int32(BLK), True, False)

    def _pair(k, base):
        base = pl.multiple_of(base, 16)
        _block(0, base, False, False)
        _block(1, base + jnp.int32(BLK), False, False)
        return base + jnp.int32(2 * BLK)
    lax.fori_loop(0, NPAIR - 2, _pair, jnp.int32(2 * BLK))

    tbase = jnp.int32((NBLK - 2) * BLK)
    _block(0, tbase, False, True)
    _block(1, tbase + jnp.int32(BLK), False, True)
    _wait_scatter(0)
    _wait_scatter(1)

    plsc.subcore_barrier()

    pltpu.sync_copy(acc_sh.at[pl.ds(r0, RPT)], stage)
    pltpu.sync_copy(stage, out_hbm.at[cid, pl.ds(r0, RPT)])


def kernel(x, edge_index, W1, att_src1, att_dst1, b1,
           W2, att_src2, att_dst2, b2):
    src = edge_index[0].astype(jnp.int32)
    dst = edge_index[1].astype(jnp.int32)
    fill = jnp.full((EPAD - E,), NP - 1, jnp.int32)
    src = jnp.concatenate([src, fill])
    dst = jnp.concatenate([dst, fill])
    hp1, as1, ad1, m1 = _dense1(x, W1, att_src1.reshape(1, D_HID),
                                att_dst1.reshape(1, D_HID))
    acc1 = _edge_sc(src, dst, as1, ad1, m1, hp1)
    hp2, as2, ad2, m2 = _dense2(acc1, b1.reshape(1, D_HID), W2,
                                att_src2.reshape(1, N_CLASSES),
                                att_dst2.reshape(1, N_CLASSES))
    acc2 = _edge_sc(src, dst, as2, ad2, m2, hp2)
    return _final(acc2, b2.reshape(1, N_CLASSES))


# speedup vs baseline: 1.4366x; 1.4366x over previous
import functools

import jax
import jax.numpy as jnp
from jax import lax
from jax.experimental import pallas as pl
from jax.experimental.pallas import tpu as pltpu
from jax.experimental.pallas import tpu_sc as plsc

N = 10000
E = 320000
D_IN = 128
D_HID = 8
N_CLASSES = 10
PAD = 16
ULANE = PAD - 1
NEG = 0.2

NC = 2
NS = 16
NW = NC * NS
NP = 10240
RPT = NP // NS
EPAD = NW * NP
EPW = EPAD // NW
BLK = 128
NBLK = EPW // BLK
NPAIR = NBLK // 2


def _lrelu(t):
    return jnp.maximum(t, NEG * t)


def _dense1_body(x_ref, w_ref, asrc_ref, adst_ref, hp_ref, as_ref, ad_ref, m_ref):
    h = jnp.dot(x_ref[...], w_ref[...], preferred_element_type=jnp.float32)
    a_s = jnp.sum(h * asrc_ref[...], axis=1)
    a_d = jnp.sum(h * adst_ref[...], axis=1)
    m = _lrelu(jnp.max(a_s) + a_d)
    hp = jnp.concatenate([h, jnp.zeros((N, PAD - D_HID), jnp.float32)], axis=1)
    hp_ref[...] = jnp.concatenate([hp, jnp.zeros((NP - N, PAD), jnp.float32)], axis=0)
    zpad = jnp.zeros((NP - N,), jnp.float32)
    as_ref[...] = jnp.concatenate([a_s, zpad])
    ad_ref[...] = jnp.concatenate([a_d, zpad])
    m_ref[...] = jnp.concatenate([m, zpad])


def _dense2_body(acc_ref, b1_ref, w2_ref, asrc_ref, adst_ref, hp_ref, as_ref, ad_ref, m_ref):
    acc = acc_ref[0] + acc_ref[1]
    o = acc[:N, :D_HID] / (acc[:N, ULANE:ULANE + 1] + 1e-16) + b1_ref[...]
    x2 = jnp.maximum(o, 0.0)
    h = jnp.dot(x2, w2_ref[...], preferred_element_type=jnp.float32)
    a_s = jnp.sum(h * asrc_ref[...], axis=1)
    a_d = jnp.sum(h * adst_ref[...], axis=1)
    m = _lrelu(jnp.max(a_s) + a_d)
    hp = jnp.concatenate([h, jnp.zeros((N, PAD - N_CLASSES), jnp.float32)], axis=1)
    hp_ref[...] = jnp.concatenate([hp, jnp.zeros((NP - N, PAD), jnp.float32)], axis=0)
    zpad = jnp.zeros((NP - N,), jnp.float32)
    as_ref[...] = jnp.concatenate([a_s, zpad])
    ad_ref[...] = jnp.concatenate([a_d, zpad])
    m_ref[...] = jnp.concatenate([m, zpad])


def _final_body(acc_ref, b2_ref, out_ref):
    acc = acc_ref[0] + acc_ref[1]
    h = acc[:N, :N_CLASSES] / (acc[:N, ULANE:ULANE + 1] + 1e-16) + b2_ref[...]
    m = jnp.max(h, axis=1, keepdims=True)
    lse = jnp.log(jnp.sum(jnp.exp(h - m), axis=1, keepdims=True))
    out_ref[...] = h - m - lse


_dense1 = pl.pallas_call(
    _dense1_body,
    out_shape=[jax.ShapeDtypeStruct((NP, PAD), jnp.float32),
               jax.ShapeDtypeStruct((NP,), jnp.float32),
               jax.ShapeDtypeStruct((NP,), jnp.float32),
               jax.ShapeDtypeStruct((NP,), jnp.float32)])

_dense2 = pl.pallas_call(
    _dense2_body,
    out_shape=[jax.ShapeDtypeStruct((NP, PAD), jnp.float32),
               jax.ShapeDtypeStruct((NP,), jnp.float32),
               jax.ShapeDtypeStruct((NP,), jnp.float32),
               jax.ShapeDtypeStruct((NP,), jnp.float32)])

_final = pl.pallas_call(
    _final_body,
    out_shape=jax.ShapeDtypeStruct((N, N_CLASSES), jnp.float32))


_sc_mesh = plsc.VectorSubcoreMesh(core_axis_name="c", subcore_axis_name="s")


@functools.partial(
    pl.kernel,
    out_type=jax.ShapeDtypeStruct((NC, NP, PAD), jnp.float32),
    mesh=_sc_mesh,
    compiler_params=pltpu.CompilerParams(needs_layout_passes=False,
                                         use_tc_tiling_on_sc=False),
    scratch_types=[
        pltpu.VMEM_SHARED((NP, PAD), jnp.float32),
        pltpu.VMEM((RPT, PAD), jnp.float32),
        pltpu.VMEM((EPW,), jnp.int32),
        pltpu.VMEM((EPW,), jnp.int32),
        pltpu.VMEM((NP,), jnp.float32),
        pltpu.VMEM((NP,), jnp.float32),
        pltpu.VMEM((NP,), jnp.float32),
        pltpu.VMEM((BLK, PAD), jnp.float32),
        pltpu.VMEM((BLK, PAD), jnp.float32),
        pltpu.VMEM((BLK, PAD), jnp.float32),
        pltpu.VMEM((BLK, PAD), jnp.float32),
        pltpu.VMEM((BLK,), jnp.float32),
        pltpu.VMEM((BLK,), jnp.int32),
        pltpu.VMEM((BLK,), jnp.int32),
        pltpu.VMEM((BLK,), jnp.int32),
        pltpu.VMEM((BLK,), jnp.int32),
        pltpu.VMEM((BLK,), jnp.int32),
        pltpu.VMEM((BLK,), jnp.int32),
        pltpu.SemaphoreType.DMA,
        pltpu.SemaphoreType.DMA,
        pltpu.SemaphoreType.DMA,
        pltpu.SemaphoreType.DMA,
    ])
def _edge_sc(src_hbm, dst_hbm, as_hbm, ad_hbm, m_hbm, hp_hbm, out_hbm,
             acc_sh, stage, srcv, dstv, asv, adv, mv,
             hr0, hr1, sr0, sr1, ublk,
             sidx0, sidx1, didx0, didx1, didxs0, didxs1,
             sg0, sg1, ss0, ss1):
    cid = lax.axis_index("c")
    sid = lax.axis_index("s")
    wid = cid * jnp.int32(NS) + sid
    r0 = sid * jnp.int32(RPT)

    sidx = (sidx0, sidx1)
    didx = (didx0, didx1)
    didxs = (didxs0, didxs1)
    hr = (hr0, hr1)
    sr = (sr0, sr1)
    sg = (sg0, sg1)
    ss = (ss0, ss1)
    lane = lax.iota(jnp.int32, 16)
    mask_u = lane == ULANE

    e0 = wid * jnp.int32(EPW)
    pltpu.async_copy(as_hbm, asv, sg0)
    pltpu.async_copy(ad_hbm, adv, sg0)
    pltpu.async_copy(m_hbm, mv, sg0)
    pltpu.async_copy(src_hbm.at[pl.ds(e0, EPW)], srcv, sg1)
    pltpu.async_copy(dst_hbm.at[pl.ds(e0, EPW)], dstv, sg1)

    def _zrow(r, carry):
        stage[r, :] = jnp.zeros((PAD,), jnp.float32)
        return carry
    lax.fori_loop(0, RPT, _zrow, 0)
    pltpu.sync_copy(stage, acc_sh.at[pl.ds(r0, RPT)])

    pltpu.make_async_copy(as_hbm, asv, sg0).wait()
    pltpu.make_async_copy(ad_hbm, adv, sg0).wait()
    pltpu.make_async_copy(m_hbm, mv, sg0).wait()
    pltpu.make_async_copy(src_hbm.at[pl.ds(e0, EPW)], srcv, sg1).wait()
    pltpu.make_async_copy(dst_hbm.at[pl.ds(e0, EPW)], dstv, sg1).wait()

    plsc.subcore_barrier()

    def _prep(p, base):
        for v in range(BLK // 16):
            sidx[p][pl.ds(v * 16, 16)] = srcv[pl.ds(base + v * 16, 16)]
            didx[p][pl.ds(v * 16, 16)] = dstv[pl.ds(base + v * 16, 16)]

    def _issue_gather(p):
        pltpu.async_copy(hp_hbm.at[sidx[p]], hr[p], sg[p])

    def _wait_gather(p):
        pltpu.make_async_copy(hp_hbm.at[sidx[p]], hr[p], sg[p]).wait()

    def _issue_scatter(p):
        pltpu.async_copy(sr[p], acc_sh.at[didxs[p]], ss[p], add=True)

    def _wait_scatter(p):
        pltpu.make_async_copy(sr[p], acc_sh.at[didxs[p]], ss[p]).wait()

    def _scale_rows(p):
        for v in range(BLK // 16):
            sv = sidx[p][pl.ds(v * 16, 16)]
            dv = didx[p][pl.ds(v * 16, 16)]
            a = plsc.load_gather(asv, [sv]) + plsc.load_gather(adv, [dv])
            a = _lrelu(a) - plsc.load_gather(mv, [dv])
            uvec = jnp.exp(a)
            for i in range(16):
                r = v * 16 + i
                ubc = uvec[jnp.full((16,), i, jnp.int32)]
                sr[p][r, :] = jnp.where(mask_u, ubc, hr[p][r, :] * ubc)

    def _copy_didx(p):
        for v in range(BLK // 16):
            didxs[p][pl.ds(v * 16, 16)] = didx[p][pl.ds(v * 16, 16)]

    def _block(p, base, first, last):
        _wait_gather(p)
        if not first:
            _wait_scatter(p)
        _scale_rows(p)
        _copy_didx(p)
        _issue_scatter(p)
        if not last:
            _prep(p, base + jnp.int32(2 * BLK))
            _issue_gather(p)

    _prep(0, jnp.int32(0))
    _issue_gather(0)
    _prep(1, jnp.int32(BLK))
    _issue_gather(1)
    _block(0, jnp.int32(0), True, False)
    _block(1, jnp.int32(BLK), True, False)

    def _pair(k, base):
        base = pl.multiple_of(base, 16)
        _block(0, base, False, False)
        _block(1, base + jnp.int32(BLK), False, False)
        return base + jnp.int32(2 * BLK)
    lax.fori_loop(0, NPAIR - 2, _pair, jnp.int32(2 * BLK))

    tbase = jnp.int32((NBLK - 2) * BLK)
    _block(0, tbase, False, True)
    _block(1, tbase + jnp.int32(BLK), False, True)
    _wait_scatter(0)
    _wait_scatter(1)

    plsc.subcore_barrier()

    pltpu.sync_copy(acc_sh.at[pl.ds(r0, RPT)], stage)
    pltpu.sync_copy(stage, out_hbm.at[cid, pl.ds(r0, RPT)])


def kernel(x, edge_index, W1, att_src1, att_dst1, b1,
           W2, att_src2, att_dst2, b2):
    src = edge_index[0].astype(jnp.int32)
    dst = edge_index[1].astype(jnp.int32)
    fill = jnp.full((EPAD - E,), NP - 1, jnp.int32)
    src = jnp.concatenate([src, fill])
    dst = jnp.concatenate([dst, fill])
    hp1, as1, ad1, m1 = _dense1(x, W1, att_src1.reshape(1, D_HID),
                                att_dst1.reshape(1, D_HID))
    acc1 = _edge_sc(src, dst, as1, ad1, m1, hp1)
    hp2, as2, ad2, m2 = _dense2(acc1, b1.reshape(1, D_HID), W2,
                                att_src2.reshape(1, N_CLASSES),
                                att_dst2.reshape(1, N_CLASSES))
    acc2 = _edge_sc(src, dst, as2, ad2, m2, hp2)
    return _final(acc2, b2.reshape(1, N_CLASSES))
